# Initial kernel scaffold; baseline (speedup 1.0000x reference)
import jax
import jax.numpy as jnp
from jax.experimental import pallas as pl


def kernel(inputs, sparse_adj_indices, Wq, bq, Wk, bk):
    raise NotImplementedError("write your pallas kernel here")


# speedup vs baseline: 4.3316x; 4.3316x over previous
import functools

import jax
import jax.numpy as jnp
from jax import lax
from jax.experimental import pallas as pl
from jax.experimental.pallas import tpu as pltpu
from jax.experimental.pallas import tpu_sc as plsc

N = 10000
E = 320000
D = 128
DW = 128

NC = 2
NS = 16
NW = NC * NS
EPW = E // NW
CHUNK = 80
NCHUNK = EPW // CHUNK
GROUPS = CHUNK // 16


def _qk_body(x_ref, wqt_ref, bq_ref, wkt_ref, bk_ref, q_ref, k_ref):
    x = x_ref[...]
    inv_dk = 1.0 / (DW ** 0.5)
    q = jax.lax.dot_general(x, wqt_ref[...], (((1,), (0,)), ((), ())),
                            preferred_element_type=jnp.float32,
                            precision=jax.lax.Precision.HIGHEST)
    k = jax.lax.dot_general(x, wkt_ref[...], (((1,), (0,)), ((), ())),
                            preferred_element_type=jnp.float32,
                            precision=jax.lax.Precision.HIGHEST)
    q_ref[...] = (q + bq_ref[...]) * inv_dk
    k_ref[...] = k + bk_ref[...]


def _tc_qk(inputs, WqT, bq2, WkT, bk2):
    return pl.pallas_call(
        _qk_body,
        out_shape=(
            jax.ShapeDtypeStruct((N, DW), jnp.float32),
            jax.ShapeDtypeStruct((N, DW), jnp.float32),
        ),
    )(inputs, WqT, bq2, WkT, bk2)


def _sc_scores_body(q_hbm, k_hbm, src_hbm, dst_hbm, out_hbm,
                    src_v, dst_v, qr, kr, pbuf, out_v, sem_q, sem_k):
    wid = lax.axis_index("s") * NC + lax.axis_index("c")
    base = wid * EPW
    pltpu.sync_copy(src_hbm.at[pl.ds(base, EPW)], src_v)
    pltpu.sync_copy(dst_hbm.at[pl.ds(base, EPW)], dst_v)

    lanes16 = lax.iota(jnp.int32, 16) * 16

    def chunk_body(i, carry):
        cb = i * CHUNK
        cq = pltpu.async_copy(q_hbm.at[src_v.at[pl.ds(cb, CHUNK)]], qr, sem_q)
        ck = pltpu.async_copy(k_hbm.at[dst_v.at[pl.ds(cb, CHUNK)]], kr, sem_k)
        cq.wait()
        ck.wait()

        def group_body(g, carry2):
            eb = g * 16
            for l in range(16):
                e = eb + l
                acc = qr[e, pl.ds(0, 16)] * kr[e, pl.ds(0, 16)]
                for j in range(1, 8):
                    acc = acc + qr[e, pl.ds(j * 16, 16)] * kr[e, pl.ds(j * 16, 16)]
                pbuf[pl.ds(l * 16, 16)] = acc
            o = jnp.zeros((16,), jnp.float32)
            for c in range(16):
                o = o + plsc.load_gather(pbuf, [lanes16 + c])
            out_v[pl.ds(cb + eb, 16)] = o
            return carry2

        lax.fori_loop(0, GROUPS, group_body, 0)
        return carry

    lax.fori_loop(0, NCHUNK, chunk_body, 0)
    pltpu.sync_copy(out_v, out_hbm.at[pl.ds(base, EPW)])


_sc_scores = functools.partial(
    pl.kernel,
    mesh=plsc.VectorSubcoreMesh(core_axis_name="c", subcore_axis_name="s"),
    out_type=jax.ShapeDtypeStruct((E,), jnp.float32),
    compiler_params=pltpu.CompilerParams(needs_layout_passes=False),
    scratch_types=[
        pltpu.VMEM((EPW,), jnp.int32),
        pltpu.VMEM((EPW,), jnp.int32),
        pltpu.VMEM((CHUNK, DW), jnp.float32),
        pltpu.VMEM((CHUNK, DW), jnp.float32),
        pltpu.VMEM((256,), jnp.float32),
        pltpu.VMEM((EPW,), jnp.float32),
        pltpu.SemaphoreType.DMA,
        pltpu.SemaphoreType.DMA,
    ],
)(_sc_scores_body)


def kernel(inputs, sparse_adj_indices, Wq, bq, Wk, bk):
    q, k = _tc_qk(inputs, Wq.T, bq.reshape(1, DW), Wk.T, bk.reshape(1, DW))
    src = sparse_adj_indices[0]
    dst = sparse_adj_indices[1]
    return _sc_scores(q, k, src, dst)


# speedup vs baseline: 7.2810x; 1.6809x over previous
import functools

import jax
import jax.numpy as jnp
from jax import lax
from jax.experimental import pallas as pl
from jax.experimental.pallas import tpu as pltpu
from jax.experimental.pallas import tpu_sc as plsc

N = 10000
E = 320000
D = 128
DW = 128

NC = 2
NS = 16
NW = NC * NS
EPW = E // NW
CHUNK = 80
NCHUNK = EPW // CHUNK
GROUPS = CHUNK // 16


def _qk_body(x_ref, wqt_ref, bq_ref, wkt_ref, bk_ref, q_ref, k_ref):
    x = x_ref[...]
    inv_dk = 1.0 / (DW ** 0.5)
    q = jax.lax.dot_general(x, wqt_ref[...], (((1,), (0,)), ((), ())),
                            preferred_element_type=jnp.float32,
                            precision=jax.lax.Precision.HIGHEST)
    k = jax.lax.dot_general(x, wkt_ref[...], (((1,), (0,)), ((), ())),
                            preferred_element_type=jnp.float32,
                            precision=jax.lax.Precision.HIGHEST)
    q_ref[...] = (q + bq_ref[...]) * inv_dk
    k_ref[...] = k + bk_ref[...]


def _tc_qk(inputs, WqT, bq2, WkT, bk2):
    return pl.pallas_call(
        _qk_body,
        out_shape=(
            jax.ShapeDtypeStruct((N, DW), jnp.float32),
            jax.ShapeDtypeStruct((N, DW), jnp.float32),
        ),
    )(inputs, WqT, bq2, WkT, bk2)


def _sc_scores_body(q_hbm, k_hbm, src_hbm, dst_hbm, out_hbm,
                    src_v, dst_v, qr0, kr0, qr1, kr1, pbuf, out_v,
                    sem_q0, sem_k0, sem_q1, sem_k1):
    wid = lax.axis_index("s") * NC + lax.axis_index("c")
    base = wid * EPW
    pltpu.sync_copy(src_hbm.at[pl.ds(base, EPW)], src_v)
    pltpu.sync_copy(dst_hbm.at[pl.ds(base, EPW)], dst_v)

    lanes16 = lax.iota(jnp.int32, 16) * 16
    bufs = ((qr0, kr0, sem_q0, sem_k0), (qr1, kr1, sem_q1, sem_k1))

    def issue(c, b):
        qr, kr, sem_q, sem_k = bufs[b]
        cb = c * CHUNK
        pltpu.async_copy(q_hbm.at[src_v.at[pl.ds(cb, CHUNK)]], qr, sem_q)
        pltpu.async_copy(k_hbm.at[dst_v.at[pl.ds(cb, CHUNK)]], kr, sem_k)

    def wait(b):
        qr, kr, sem_q, sem_k = bufs[b]
        pltpu.make_async_copy(q_hbm.at[src_v.at[pl.ds(0, CHUNK)]], qr, sem_q).wait()
        pltpu.make_async_copy(k_hbm.at[dst_v.at[pl.ds(0, CHUNK)]], kr, sem_k).wait()

    def compute(c, b):
        qr, kr, _, _ = bufs[b]
        cb = c * CHUNK

        def group_body(g, carry2):
            eb = g * 16
            for l in range(16):
                e = eb + l
                acc = qr[e, pl.ds(0, 16)] * kr[e, pl.ds(0, 16)]
                for j in range(1, 8):
                    acc = acc + qr[e, pl.ds(j * 16, 16)] * kr[e, pl.ds(j * 16, 16)]
                pbuf[pl.ds(l * 16, 16)] = acc
            o = jnp.zeros((16,), jnp.float32)
            for c2 in range(16):
                o = o + plsc.load_gather(pbuf, [lanes16 + c2])
            out_v[pl.ds(cb + eb, 16)] = o
            return carry2

        lax.fori_loop(0, GROUPS, group_body, 0)

    issue(0, 0)

    def pair_body(i, carry):
        c = 2 * i
        wait(0)
        issue(c + 1, 1)
        compute(c, 0)
        wait(1)
        issue(c + 2, 0)
        compute(c + 1, 1)
        return carry

    lax.fori_loop(0, (NCHUNK - 1) // 2, pair_body, 0)
    wait(0)
    compute(NCHUNK - 1, 0)
    pltpu.sync_copy(out_v, out_hbm.at[pl.ds(base, EPW)])


_sc_scores = functools.partial(
    pl.kernel,
    mesh=plsc.VectorSubcoreMesh(core_axis_name="c", subcore_axis_name="s"),
    out_type=jax.ShapeDtypeStruct((E,), jnp.float32),
    compiler_params=pltpu.CompilerParams(needs_layout_passes=False),
    scratch_types=[
        pltpu.VMEM((EPW,), jnp.int32),
        pltpu.VMEM((EPW,), jnp.int32),
        pltpu.VMEM((CHUNK, DW), jnp.float32),
        pltpu.VMEM((CHUNK, DW), jnp.float32),
        pltpu.VMEM((CHUNK, DW), jnp.float32),
        pltpu.VMEM((CHUNK, DW), jnp.float32),
        pltpu.VMEM((256,), jnp.float32),
        pltpu.VMEM((EPW,), jnp.float32),
        pltpu.SemaphoreType.DMA,
        pltpu.SemaphoreType.DMA,
        pltpu.SemaphoreType.DMA,
        pltpu.SemaphoreType.DMA,
    ],
)(_sc_scores_body)


def kernel(inputs, sparse_adj_indices, Wq, bq, Wk, bk):
    q, k = _tc_qk(inputs, Wq.T, bq.reshape(1, DW), Wk.T, bk.reshape(1, DW))
    src = sparse_adj_indices[0]
    dst = sparse_adj_indices[1]
    return _sc_scores(q, k, src, dst)
